# baseline (speedup 1.0000x reference)
import functools

import jax
import jax.numpy as jnp
import numpy as np
from jax import lax
from jax.experimental import pallas as pl
from jax.experimental.pallas import tpu as pltpu
from jax.experimental.pallas import tpu_sc as plsc

_FILLER = np.float32(10.1)

_B = 16
_IR = 2048
_IC = 512
_N = _IR * _IC
_CH = 8192
_CR = _CH // _IC
_NCHUNK = _N // _CH
_L = 16
_RW = 4096
_RB = _RW // _IC
_SEG = _N // 16
_NRECT = _SEG // _RW


def _make_compact():
    mesh = plsc.VectorSubcoreMesh(core_axis_name="c", subcore_axis_name="s")

    @functools.partial(
        pl.kernel,
        mesh=mesh,
        out_type=(
            jax.ShapeDtypeStruct((_B, _N), jnp.float32),
            jax.ShapeDtypeStruct((_B * _N,), jnp.float32),
        ),
        compiler_params=pltpu.CompilerParams(needs_layout_passes=False),
        scratch_types=[
            pltpu.VMEM((_CR, _IC), jnp.float32),
            pltpu.VMEM((_CR, _IC), jnp.float32),
            pltpu.VMEM((2 * _CH + 2 * _L,), jnp.float32),
            pltpu.VMEM((8, _RB, _IC), jnp.float32),
            pltpu.VMEM((8, _RB, _IC), jnp.float32),
            pltpu.VMEM((_L,), jnp.int32),
            pltpu.VMEM((16 * _L,), jnp.int32),
            pltpu.VMEM_SHARED((16 * _L,), jnp.int32),
            pltpu.SemaphoreType.DMA,
            pltpu.SemaphoreType.DMA,
            pltpu.SemaphoreType.DMA,
            pltpu.SemaphoreType.DMA,
            pltpu.SemaphoreType.DMA,
            pltpu.SemaphoreType.DMA,
        ],
    )
    def compact(x_hbm, out_hbm, scr_hbm, buf0, buf1, stage,
                rect0, rect1, bounce, allcnt, counts_sh,
                is0, is1, ri0, ri1, ro0, ro1):
        bufs = (buf0, buf1)
        isems = (is0, is1)
        rects = (rect0, rect1)
        risems = (ri0, ri1)
        rosems = (ro0, ro1)
        c = lax.axis_index("c")
        s = lax.axis_index("s")

        rbase = c * 8
        colbase = s * _SEG
        iota16 = lax.iota(jnp.int32, _L)

        def rect_in_wait(b):
            for r in range(8):
                pltpu.make_async_copy(x_hbm.at[0, pl.ds(0, _RB), :],
                                      rects[b].at[r], risems[b]).wait()

        def rect_out(t, b):
            co = colbase + t * _RW
            for u in range(_RB):
                pltpu.async_copy(
                    rects[b].at[:, u, :],
                    out_hbm.at[pl.ds(pl.multiple_of(rbase, 8), 8),
                               pl.ds(pl.multiple_of(co + u * _IC, _IC),
                                     _IC)],
                    rosems[b])

        def rect_out_wait(b):
            for u in range(_RB):
                pltpu.make_async_copy(
                    rects[b].at[:, u, :],
                    out_hbm.at[pl.ds(0, 8), pl.ds(0, _IC)],
                    rosems[b]).wait()

        def rect_in_x(t, b):
            it = pl.multiple_of((colbase + t * _RW) // _IC, _RB)
            for r in range(8):
                pltpu.async_copy(x_hbm.at[rbase + r, pl.ds(it, _RB), :],
                                 rects[b].at[r], risems[b])

        def count_rect(rect, accs):
            def cb(j, a):
                a = list(a)
                for r in range(8):
                    for jr in range(_RB):
                        v = rect[r, jr, pl.ds(j * _L, _L)]
                        a[r] = a[r] + jnp.where(v != _FILLER, jnp.int32(1),
                                                jnp.int32(0))
                return tuple(a)

            return lax.fori_loop(0, _IC // _L, cb, accs)

        rect_in_x(0, 0)
        zero = jnp.zeros((_L,), jnp.int32)

        @pl.loop(0, _NRECT, step=2, init_carry=tuple(zero for _ in range(8)))
        def accs(t0, accs_c):
            for p in range(2):
                t = t0 + p
                b = p
                nb = 1 - p
                rect_in_wait(b)
                rect_out(t, b)
                accs_c = count_rect(rects[b], accs_c)

                @pl.when(t + 1 < _NRECT)
                def _():
                    @pl.when(t >= 1)
                    def _():
                        rect_out_wait(nb)
                    rect_in_x(t + 1, nb)
            return accs_c

        rect_out_wait(0)
        rect_out_wait(1)

        pack = jnp.zeros((_L,), jnp.int32)
        for r in range(8):
            pack = jnp.where(iota16 == r, jnp.sum(accs[r]), pack)
        bounce[pl.ds(0, _L)] = pack
        pltpu.sync_copy(bounce, counts_sh.at[pl.ds(s * _L, _L)])
        plsc.subcore_barrier()

        pltpu.sync_copy(counts_sh, allcnt)
        total_vec = jnp.zeros((_L,), jnp.int32)
        for i in range(16):
            total_vec = total_vec + allcnt[pl.ds(i * _L, _L)]
        flags = [total_vec[r] == _N for r in range(8)]
        clean = flags[0]
        for r in range(1, 8):
            clean = jnp.logical_and(clean, flags[r])
        any_dirty = jnp.logical_not(clean)

        @pl.when(jnp.logical_and(s < 8, any_dirty))
        def _phaseB():
            row = c * 8 + s
            base = pl.multiple_of(row * _N, _CH)
            dirty_mine = jnp.logical_not(
                jnp.sum(jnp.where(iota16 == s, total_vec, 0)) == _N)

            @pl.when(dirty_mine)
            def _compact_row():
                def out_slice(woff):
                    return scr_hbm.at[
                        pl.ds(pl.multiple_of(base + woff, _CH), _CH)]

                def issue_in(k, b):
                    ir0 = pl.multiple_of(k * _CR, _CR)
                    pltpu.async_copy(x_hbm.at[row, pl.ds(ir0, _CR), :],
                                     bufs[b], isems[b])

                def wait_in(b):
                    pltpu.make_async_copy(x_hbm.at[0, pl.ds(0, _CR), :],
                                          bufs[b], isems[b]).wait()

                def make_inner(buf, i):
                    def inner(j, fill):
                        v = buf[i, pl.ds(j * _L, _L)]
                        m = v != _FILLER
                        plsc.store_compressed(stage.at[pl.ds(fill, _L)], v,
                                              mask=m)
                        cnt = plsc.all_reduce_population_count(m)[0]
                        return fill + cnt
                    return inner

                issue_in(0, 0)

                def process(k, b, carry):
                    fill, woff = carry
                    nb = 1 - b

                    if b == 1:
                        @pl.when(k + 1 < _NCHUNK)
                        def _():
                            issue_in(k + 1, nb)
                    else:
                        issue_in(k + 1, nb)

                    wait_in(b)

                    for i in range(_CR):
                        fill = lax.fori_loop(0, _IC // _L,
                                             make_inner(bufs[b], i), fill)

                    @pl.when(fill >= _CH)
                    def _flush():
                        pltpu.sync_copy(stage.at[pl.ds(0, _CH)],
                                        out_slice(woff))
                        nmove = (fill - _CH + _L - 1) // _L

                        def mv(j, _):
                            stage[pl.ds(j * _L, _L)] = (
                                stage[pl.ds(_CH + j * _L, _L)])
                            return 0

                        lax.fori_loop(0, nmove, mv, 0)

                    do = fill >= _CH
                    fill = jnp.where(do, fill - _CH, fill)
                    woff = jnp.where(do, woff + _CH, woff)
                    return fill, woff

                init = (jnp.int32(0), jnp.int32(0))

                @pl.loop(0, _NCHUNK, step=2, init_carry=init)
                def group(k0, carry):
                    for b in range(2):
                        carry = process(k0 + b, b, carry)
                    return carry

                fill, woff = group

                @pl.when(woff < _N)
                def _tail():
                    def pad(j, _):
                        idx = j * _L + iota16
                        v = stage[pl.ds(j * _L, _L)]
                        stage[pl.ds(j * _L, _L)] = jnp.where(
                            idx >= fill, _FILLER, v)
                        return 0

                    lax.fori_loop(0, _CH // _L, pad, 0)
                    pltpu.sync_copy(stage.at[pl.ds(0, _CH)],
                                    out_slice(woff))

                    def fillstage(j, _):
                        stage[pl.ds(j * _L, _L)] = jnp.full((_L,), _FILLER)
                        return 0

                    lax.fori_loop(0, _CH // _L, fillstage, 0)

                    def more(woff2):
                        pltpu.sync_copy(stage.at[pl.ds(0, _CH)],
                                        out_slice(woff2))
                        return woff2 + _CH

                    lax.while_loop(lambda w2: w2 < _N, more, woff + _CH)

        @pl.when(any_dirty)
        def _phaseC():
            plsc.subcore_barrier()

            def rect_in_mixed(t, b):
                it = pl.multiple_of((colbase + t * _RW) // _IC, _RB)
                co = colbase + t * _RW
                for r in range(8):
                    @pl.when(flags[r])
                    def _(r=r, b=b):
                        pltpu.async_copy(
                            x_hbm.at[rbase + r, pl.ds(it, _RB), :],
                            rects[b].at[r], risems[b])

                    @pl.when(jnp.logical_not(flags[r]))
                    def _(r=r, b=b):
                        off = pl.multiple_of((rbase + r) * _N + co, _IC)
                        for jr in range(_RB):
                            pltpu.async_copy(
                                scr_hbm.at[pl.ds(off + jr * _IC, _IC)],
                                rects[b].at[r, jr], risems[b])

            rect_in_mixed(0, 0)

            @pl.loop(0, _NRECT, step=2)
            def rloop(t0):
                for p in range(2):
                    t = t0 + p
                    b = p
                    nb = 1 - p
                    rect_in_wait(b)
                    rect_out(t, b)

                    @pl.when(t + 1 < _NRECT)
                    def _():
                        @pl.when(t >= 1)
                        def _():
                            rect_out_wait(nb)
                        rect_in_mixed(t + 1, nb)

            rect_out_wait(0)
            rect_out_wait(1)

    return compact


_compact = _make_compact()


def kernel(x1, x2, x3):
    out, _scr = _compact(x3)
    return out


# speedup vs baseline: 2.8175x; 2.8175x over previous
import functools

import jax
import jax.numpy as jnp
import numpy as np
from jax import lax
from jax.experimental import pallas as pl
from jax.experimental.pallas import tpu as pltpu
from jax.experimental.pallas import tpu_sc as plsc

_FILLER = np.float32(10.1)

_B = 16
_N = 2048 * 512
_CH = 8192
_NCHUNK = _N // _CH
_L = 16
_CU = 8
_RW = 4096
_SEG = _N // 16
_NRECT = _SEG // _RW


def _make_compact():
    mesh = plsc.VectorSubcoreMesh(core_axis_name="c", subcore_axis_name="s")

    @functools.partial(
        pl.kernel,
        mesh=mesh,
        out_type=(
            jax.ShapeDtypeStruct((_B, _N), jnp.float32),
            jax.ShapeDtypeStruct((_B * _N,), jnp.float32),
        ),
        compiler_params=pltpu.CompilerParams(needs_layout_passes=False),
        scratch_types=[
            pltpu.VMEM((_CH,), jnp.float32),
            pltpu.VMEM((_CH,), jnp.float32),
            pltpu.VMEM((2 * _CH + 2 * _L,), jnp.float32),
            pltpu.VMEM((8, _RW), jnp.float32),
            pltpu.VMEM((8, _RW), jnp.float32),
            pltpu.VMEM((_L,), jnp.int32),
            pltpu.VMEM((16 * _L,), jnp.int32),
            pltpu.VMEM_SHARED((16 * _L,), jnp.int32),
            pltpu.SemaphoreType.DMA,
            pltpu.SemaphoreType.DMA,
            pltpu.SemaphoreType.DMA,
            pltpu.SemaphoreType.DMA,
            pltpu.SemaphoreType.DMA,
            pltpu.SemaphoreType.DMA,
            pltpu.SemaphoreType.DMA,
            pltpu.SemaphoreType.DMA,
        ],
    )
    def compact(x_hbm, out_hbm, scr_hbm, buf0, buf1, stage,
                rect0, rect1, bounce, allcnt, counts_sh,
                is0, is1, os0, os1, ri0, ri1, ro0, ro1):
        bufs = (buf0, buf1)
        isems = (is0, is1)
        osems = (os0, os1)
        rects = (rect0, rect1)
        risems = (ri0, ri1)
        rosems = (ro0, ro1)
        c = lax.axis_index("c")
        s = lax.axis_index("s")

        rbase = c * 8
        colbase = s * _SEG
        iota16 = lax.iota(jnp.int32, _L)

        def row_off(r, t):
            return pl.multiple_of((rbase + r) * _N + colbase + t * _RW, _RW)

        def rect_in_wait(b):
            for r in range(8):
                pltpu.make_async_copy(scr_hbm.at[pl.ds(0, _RW)],
                                      rects[b].at[r], risems[b]).wait()

        def rect_out(t, b):
            pltpu.async_copy(
                rects[b],
                out_hbm.at[pl.ds(pl.multiple_of(rbase, 8), 8),
                           pl.ds(pl.multiple_of(colbase + t * _RW, _RW),
                                 _RW)],
                rosems[b])

        def rect_out_wait(b):
            pltpu.make_async_copy(
                rects[b],
                out_hbm.at[pl.ds(0, 8), pl.ds(0, _RW)],
                rosems[b]).wait()

        def rect_in_x(t, b):
            for r in range(8):
                pltpu.async_copy(x_hbm.at[pl.ds(row_off(r, t), _RW)],
                                 rects[b].at[r], risems[b])

        def count_rect(rect, accs):
            def cb(j, a):
                a = list(a)
                for r in range(8):
                    v = rect[r, pl.ds(j * _L, _L)]
                    a[r] = a[r] + jnp.where(v != _FILLER, jnp.int32(1),
                                            jnp.int32(0))
                return tuple(a)

            return lax.fori_loop(0, _RW // _L, cb, accs)

        rect_in_x(0, 0)
        zero = jnp.zeros((_L,), jnp.int32)

        @pl.loop(0, _NRECT, step=2, init_carry=tuple(zero for _ in range(8)))
        def accs(t0, accs_c):
            for p in range(2):
                t = t0 + p
                b = p
                nb = 1 - p
                rect_in_wait(b)
                rect_out(t, b)
                accs_c = count_rect(rects[b], accs_c)

                @pl.when(t + 1 < _NRECT)
                def _():
                    @pl.when(t >= 1)
                    def _():
                        rect_out_wait(nb)
                    rect_in_x(t + 1, nb)
            return accs_c

        rect_out_wait(0)
        rect_out_wait(1)

        pack = jnp.zeros((_L,), jnp.int32)
        for r in range(8):
            pack = jnp.where(iota16 == r, jnp.sum(accs[r]), pack)
        bounce[pl.ds(0, _L)] = pack
        pltpu.sync_copy(bounce, counts_sh.at[pl.ds(s * _L, _L)])
        plsc.subcore_barrier()

        pltpu.sync_copy(counts_sh, allcnt)
        total_vec = jnp.zeros((_L,), jnp.int32)
        for i in range(16):
            total_vec = total_vec + allcnt[pl.ds(i * _L, _L)]
        flags = [total_vec[r] == _N for r in range(8)]
        clean = flags[0]
        for r in range(1, 8):
            clean = jnp.logical_and(clean, flags[r])
        any_dirty = jnp.logical_not(clean)

        @pl.when(jnp.logical_and(s < 8, any_dirty))
        def _phaseB():
            row = c * 8 + s
            base = pl.multiple_of(row * _N, _CH)
            dirty_mine = jnp.logical_not(
                jnp.sum(jnp.where(iota16 == s, total_vec, 0)) == _N)

            @pl.when(dirty_mine)
            def _compact_row():
                def in_slice(k):
                    return x_hbm.at[
                        pl.ds(pl.multiple_of(base + k * _CH, _CH), _CH)]

                def out_slice(woff):
                    return scr_hbm.at[
                        pl.ds(pl.multiple_of(base + woff, _CH), _CH)]

                def count_chunk(buf):
                    def cb(j, a4):
                        a4 = list(a4)
                        for u in range(_CU):
                            v = buf[pl.ds((j * _CU + u) * _L, _L)]
                            a4[u % 4] = a4[u % 4] + jnp.where(
                                v != _FILLER, jnp.int32(1), jnp.int32(0))
                        return tuple(a4)

                    a0, a1, a2, a3 = lax.fori_loop(
                        0, _CH // (_CU * _L), cb, (zero, zero, zero, zero))
                    return jnp.sum(a0 + a1 + a2 + a3)

                def make_inner(buf):
                    def inner(j, fill):
                        v = buf[pl.ds(j * _L, _L)]
                        m = v != _FILLER
                        plsc.store_compressed(stage.at[pl.ds(fill, _L)], v,
                                              mask=m)
                        cnt = plsc.all_reduce_population_count(m)[0]
                        return fill + cnt
                    return inner

                pltpu.async_copy(in_slice(0), bufs[0], isems[0])

                def process(k, b, carry):
                    fill, woff, pend = carry
                    nb = 1 - b
                    @pl.when(pend[nb] == 1)
                    def _():
                        pltpu.make_async_copy(bufs[nb], out_slice(0),
                                              osems[nb]).wait()
                    pend = tuple(jnp.int32(0) if i == nb else p
                                 for i, p in enumerate(pend))

                    if b == 1:
                        @pl.when(k + 1 < _NCHUNK)
                        def _():
                            pltpu.async_copy(in_slice(k + 1), bufs[nb],
                                             isems[nb])
                    else:
                        pltpu.async_copy(in_slice(k + 1), bufs[nb],
                                         isems[nb])

                    pltpu.make_async_copy(in_slice(k), bufs[b],
                                          isems[b]).wait()

                    cnt = count_chunk(bufs[b])
                    fast = jnp.logical_and(cnt == _CH, fill == 0)

                    def fast_fn(fill, woff):
                        pltpu.async_copy(bufs[b], out_slice(woff), osems[b])
                        return fill, woff + _CH, jnp.int32(1)

                    def slow_fn(fill, woff):
                        fill = lax.fori_loop(0, _CH // _L,
                                             make_inner(bufs[b]), fill)

                        @pl.when(fill >= _CH)
                        def _flush():
                            pltpu.sync_copy(stage.at[pl.ds(0, _CH)],
                                            out_slice(woff))
                            nmove = (fill - _CH + _L - 1) // _L

                            def mv(j, _):
                                stage[pl.ds(j * _L, _L)] = (
                                    stage[pl.ds(_CH + j * _L, _L)])
                                return 0

                            lax.fori_loop(0, nmove, mv, 0)

                        do = fill >= _CH
                        fill = jnp.where(do, fill - _CH, fill)
                        woff = jnp.where(do, woff + _CH, woff)
                        return fill, woff, jnp.int32(0)

                    fill, woff, pb = lax.cond(fast, fast_fn, slow_fn,
                                              fill, woff)
                    pend = tuple(pb if i == b else p
                                 for i, p in enumerate(pend))
                    return fill, woff, pend

                init = (jnp.int32(0), jnp.int32(0),
                        (jnp.int32(0), jnp.int32(0)))

                @pl.loop(0, _NCHUNK, step=2, init_carry=init)
                def group(k0, carry):
                    for b in range(2):
                        carry = process(k0 + b, b, carry)
                    return carry

                fill, woff, pend = group

                for b in range(2):
                    @pl.when(pend[b] == 1)
                    def _(b=b):
                        pltpu.make_async_copy(bufs[b], out_slice(0),
                                              osems[b]).wait()

                @pl.when(woff < _N)
                def _tail():
                    def pad(j, _):
                        idx = j * _L + iota16
                        v = stage[pl.ds(j * _L, _L)]
                        stage[pl.ds(j * _L, _L)] = jnp.where(
                            idx >= fill, _FILLER, v)
                        return 0

                    lax.fori_loop(0, _CH // _L, pad, 0)
                    pltpu.sync_copy(stage.at[pl.ds(0, _CH)],
                                    out_slice(woff))

                    def fillbuf(j, _):
                        buf0[pl.ds(j * _L, _L)] = jnp.full((_L,), _FILLER)
                        return 0

                    lax.fori_loop(0, _CH // _L, fillbuf, 0)

                    def more(woff2):
                        pltpu.sync_copy(bufs[0], out_slice(woff2))
                        return woff2 + _CH

                    lax.while_loop(lambda w2: w2 < _N, more, woff + _CH)

        @pl.when(any_dirty)
        def _phaseC():
            plsc.subcore_barrier()

            def rect_in_mixed(t, b):
                for r in range(8):
                    @pl.when(flags[r])
                    def _(r=r, b=b, t=t):
                        pltpu.async_copy(
                            x_hbm.at[pl.ds(row_off(r, t), _RW)],
                            rects[b].at[r], risems[b])

                    @pl.when(jnp.logical_not(flags[r]))
                    def _(r=r, b=b, t=t):
                        pltpu.async_copy(
                            scr_hbm.at[pl.ds(row_off(r, t), _RW)],
                            rects[b].at[r], risems[b])

            rect_in_mixed(0, 0)

            @pl.loop(0, _NRECT, step=2)
            def rloop(t0):
                for p in range(2):
                    t = t0 + p
                    b = p
                    nb = 1 - p
                    rect_in_wait(b)
                    rect_out(t, b)

                    @pl.when(t + 1 < _NRECT)
                    def _():
                        @pl.when(t >= 1)
                        def _():
                            rect_out_wait(nb)
                        rect_in_mixed(t + 1, nb)

            rect_out_wait(0)
            rect_out_wait(1)

    return compact


_compact = _make_compact()


def kernel(x1, x2, x3):
    x = x3.reshape(_B * _N)
    out, _scr = _compact(x)
    return out
